# baseline (speedup 1.0000x reference)
import functools

import jax
import jax.numpy as jnp
from jax import lax
from jax.experimental import pallas as pl
from jax.experimental.pallas import tpu as pltpu
from jax.experimental.pallas import tpu_sc as plsc

NUM_USERS = 50000
NUM_ITEMS = 50000
N = NUM_USERS + NUM_ITEMS
EMB = 32
HALF = 16
E = 1600000

NS = 16
EP = 1638400
EPT = EP // NS
C = 128
NCHUNK = EPT // C
NQ = NCHUNK // 4
PQ = EP // (4 * C)
NP = 100096
RPT = NP // NS
ZR = RPT // 17

_BLK = 1024


def _sc_propagate_body(ego_ref, pk_ref, val_ref, side_lo, side_hi,
                       q0, q1, vq0, vq1, r0, r1, r2, r3, zbuf, acc,
                       sqi0, sqi1, sqv0, sqv1,
                       sg0, sg1, sg2, sg3, ss0, ss1, ss2, ss3):
    c = lax.axis_index("c")
    s = lax.axis_index("s")
    Q, VQ = [q0, q1], [vq0, vq1]
    R = [r0, r1, r2, r3]
    SQI, SQV = [sqi0, sqi1], [sqv0, sqv1]
    SG, SS = [sg0, sg1, sg2, sg3], [ss0, ss1, ss2, ss3]

    zero16 = jnp.zeros((HALF,), jnp.float32)

    def zrow(i, carry):
        zbuf[i, :] = zero16
        return carry
    lax.fori_loop(0, ZR, zrow, 0)

    row0 = s * RPT

    def zcp(k, carry):
        rr = pl.multiple_of(row0 + k * ZR, 8)
        pltpu.sync_copy(zbuf, acc.at[pl.ds(rr, ZR)])
        return carry
    lax.fori_loop(0, RPT // ZR, zcp, 0)
    plsc.subcore_barrier()

    coff = c * NP
    qbase = s * NQ

    def qstart(qid, b):
        pltpu.async_copy(pk_ref.at[qid], Q[b], SQI[b])
        pltpu.async_copy(val_ref.at[qid], VQ[b], SQV[b])

    def qwait(b):
        pltpu.make_async_copy(pk_ref.at[0], Q[b], SQI[b]).wait()
        pltpu.make_async_copy(val_ref.at[0], VQ[b], SQV[b]).wait()

    def adjust(b, j):
        def adj(h, cr):
            o = pl.multiple_of(h * HALF, HALF)
            Q[b][j, 0, pl.ds(o, HALF)] = Q[b][j, 0, pl.ds(o, HALF)] + coff
            return cr
        lax.fori_loop(0, C // HALF, adj, 0)

    def gs(b, j, r):
        pltpu.async_copy(ego_ref.at[Q[b].at[j, 0]], R[r], SG[r])

    def gw(b, j, r):
        pltpu.make_async_copy(ego_ref.at[Q[b].at[j, 0]], R[r], SG[r]).wait()

    def scale(b, j, r):
        def sc16(h, cr):
            o = pl.multiple_of(h * HALF, HALF)
            vv = VQ[b][j, pl.ds(o, HALF)]
            for t in range(HALF):
                R[r][o + t, :] = R[r][o + t, :] * vv[t]
            return cr
        lax.fori_loop(0, C // HALF, sc16, 0)

    def st(b, j, r):
        pltpu.async_copy(R[r], acc.at[Q[b].at[j, 1]], SS[r], add=True)

    def sw(b, j, r):
        pltpu.make_async_copy(R[r], acc.at[Q[b].at[j, 1]], SS[r]).wait()

    qstart(qbase + 0, 0)
    qstart(qbase + 1, 1)
    qwait(0)
    adjust(0, 0)
    gs(0, 0, 0)
    adjust(0, 1)
    gs(0, 1, 1)

    adjust(0, 2)
    gs(0, 2, 2)
    adjust(0, 3)
    gs(0, 3, 3)
    gw(0, 0, 0); scale(0, 0, 0); st(0, 0, 0)
    gw(0, 1, 1); scale(0, 1, 1); st(0, 1, 1)
    qwait(1)
    sw(0, 0, 0); adjust(1, 0); gs(1, 0, 0)
    sw(0, 1, 1); adjust(1, 1); gs(1, 1, 1)
    gw(0, 2, 2); scale(0, 2, 2); st(0, 2, 2)
    gw(0, 3, 3); scale(0, 3, 3); st(0, 3, 3)
    qstart(qbase + 2, 0)

    def body(m_id, a, nb):
        sw(a, 2, 2); adjust(a, 2); gs(a, 2, 2)
        sw(a, 3, 3); adjust(a, 3); gs(a, 3, 3)
        gw(a, 0, 0); scale(a, 0, 0); st(a, 0, 0)
        gw(a, 1, 1); scale(a, 1, 1); st(a, 1, 1)
        qwait(nb)
        sw(a, 0, 0); adjust(nb, 0); gs(nb, 0, 0)
        sw(a, 1, 1); adjust(nb, 1); gs(nb, 1, 1)
        gw(a, 2, 2); scale(a, 2, 2); st(a, 2, 2)
        gw(a, 3, 3); scale(a, 3, 3); st(a, 3, 3)
        qstart(m_id + 2, a)

    def pair(p, carry):
        m1 = qbase + 2 * p + 1
        body(m1, 1, 0)
        body(m1 + 1, 0, 1)
        return carry
    lax.fori_loop(0, (NQ - 2) // 2, pair, 0)

    sw(1, 2, 2); adjust(1, 2); gs(1, 2, 2)
    sw(1, 3, 3); adjust(1, 3); gs(1, 3, 3)
    gw(1, 0, 0); scale(1, 0, 0); st(1, 0, 0)
    gw(1, 1, 1); scale(1, 1, 1); st(1, 1, 1)
    gw(1, 2, 2); scale(1, 2, 2); st(1, 2, 2)
    gw(1, 3, 3); scale(1, 3, 3); st(1, 3, 3)
    sw(1, 0, 0)
    sw(1, 1, 1)
    sw(1, 2, 2)
    sw(1, 3, 3)
    qwait(0)
    plsc.subcore_barrier()

    @pl.when(c == 0)
    def _():
        def wb(k, carry):
            rr = pl.multiple_of(row0 + k * ZR, 8)
            pltpu.sync_copy(acc.at[pl.ds(rr, ZR)], zbuf)
            pltpu.sync_copy(zbuf, side_lo.at[pl.ds(rr, ZR)])
            return carry
        lax.fori_loop(0, RPT // ZR, wb, 0)

    @pl.when(c == 1)
    def _():
        def wb(k, carry):
            rr = pl.multiple_of(row0 + k * ZR, 8)
            pltpu.sync_copy(acc.at[pl.ds(rr, ZR)], zbuf)
            pltpu.sync_copy(zbuf, side_hi.at[pl.ds(rr, ZR)])
            return carry
        lax.fori_loop(0, RPT // ZR, wb, 0)


_sc_propagate = pl.kernel(
    _sc_propagate_body,
    out_type=(jax.ShapeDtypeStruct((NP, HALF), jnp.float32),
              jax.ShapeDtypeStruct((NP, HALF), jnp.float32)),
    mesh=plsc.VectorSubcoreMesh(core_axis_name="c", subcore_axis_name="s"),
    compiler_params=pltpu.CompilerParams(use_tc_tiling_on_sc=False),
    scratch_types=(
        [
            pltpu.VMEM((4, 2, C), jnp.int32),
            pltpu.VMEM((4, 2, C), jnp.int32),
            pltpu.VMEM((4, C), jnp.float32),
            pltpu.VMEM((4, C), jnp.float32),
        ]
        + [pltpu.VMEM((C, HALF), jnp.float32)] * 4
        + [
            pltpu.VMEM((ZR, HALF), jnp.float32),
            pltpu.VMEM_SHARED((NP, HALF), jnp.float32),
        ]
        + [pltpu.SemaphoreType.DMA] * 12
    ),
)


def _combine_body(side2_ref, ego2_ref, wg_ref, bg_ref, wb_ref, bb_ref,
                  out2_ref, outf_ref):
    side = jnp.concatenate([side2_ref[0], side2_ref[1]], axis=1)
    ego = jnp.concatenate([ego2_ref[0], ego2_ref[1]], axis=1)
    s = jnp.dot(side, wg_ref[...], preferred_element_type=jnp.float32) + bg_ref[...]
    s = jnp.where(s >= 0, s, 0.01 * s)
    b = jnp.dot(ego * side, wb_ref[...], preferred_element_type=jnp.float32) + bb_ref[...]
    b = jnp.where(b >= 0, b, 0.01 * b)
    res = s + b
    outf_ref[...] = res
    out2_ref[0] = res[:, :HALF]
    out2_ref[1] = res[:, HALF:]


def _combine(side2, ego2, Wg, bg, Wb, bb):
    grid = (NP + _BLK - 1) // _BLK
    return pl.pallas_call(
        _combine_body,
        grid=(grid,),
        in_specs=[
            pl.BlockSpec((2, _BLK, HALF), lambda i: (0, i, 0)),
            pl.BlockSpec((2, _BLK, HALF), lambda i: (0, i, 0)),
            pl.BlockSpec((EMB, EMB), lambda i: (0, 0)),
            pl.BlockSpec((1, EMB), lambda i: (0, 0)),
            pl.BlockSpec((EMB, EMB), lambda i: (0, 0)),
            pl.BlockSpec((1, EMB), lambda i: (0, 0)),
        ],
        out_specs=[
            pl.BlockSpec((2, _BLK, HALF), lambda i: (0, i, 0)),
            pl.BlockSpec((_BLK, EMB), lambda i: (i, 0)),
        ],
        out_shape=[
            jax.ShapeDtypeStruct((2, NP, HALF), jnp.float32),
            jax.ShapeDtypeStruct((NP, EMB), jnp.float32),
        ],
    )(side2, ego2, Wg.T, bg.reshape(1, EMB), Wb.T, bb.reshape(1, EMB))


def kernel(user_indices, item_indices, adj_indices, adj_values, user_emb,
           item_emb, W_gc0, b_gc0, W_bi0, b_bi0, W_gc1, b_gc1, W_bi1, b_bi1):
    u_emb = user_emb
    i_emb = item_emb
    ego0_flat = jnp.concatenate([u_emb, i_emb], axis=0)
    zpad = jnp.zeros((NP - N, HALF), jnp.float32)
    ego_cat = jnp.concatenate(
        [u_emb[:, :HALF], i_emb[:, :HALF], zpad,
         u_emb[:, HALF:], i_emb[:, HALF:], zpad],
        axis=0)

    pad = EP - E
    src = jnp.pad(adj_indices[0], (0, pad))
    dst = jnp.pad(adj_indices[1], (0, pad))
    vals = jnp.pad(adj_values, (0, pad))
    srcr = src.reshape(PQ, 4, C)
    dstr = dst.reshape(PQ, 4, C)
    packed = jnp.pad(jnp.stack([srcr, dstr], axis=2),
                     ((0, 1), (0, 0), (0, 0), (0, 0)))
    valsr = jnp.pad(vals.reshape(PQ, 4, C), ((0, 1), (0, 0), (0, 0)))

    ego2 = ego_cat.reshape(2, NP, HALF)
    flats = [ego0_flat]
    for (Wg, bg, Wb, bb) in ((W_gc0, b_gc0, W_bi0, b_bi0),
                             (W_gc1, b_gc1, W_bi1, b_bi1)):
        side_lo, side_hi = _sc_propagate(ego2.reshape(2 * NP, HALF),
                                         packed, valsr)
        side2 = jnp.stack([side_lo, side_hi])
        ego2, ego_flat = _combine(side2, ego2, Wg, bg, Wb, bb)
        flats.append(ego_flat[:N])
    all_c = jnp.concatenate(flats, axis=1)
    return (all_c[:NUM_USERS], all_c[NUM_USERS:])


# speedup vs baseline: 1.1163x; 1.1163x over previous
import jax
import jax.numpy as jnp
from jax import lax
from jax.experimental import pallas as pl
from jax.experimental.pallas import tpu as pltpu
from jax.experimental.pallas import tpu_sc as plsc

NUM_USERS = 50000
NUM_ITEMS = 50000
N = NUM_USERS + NUM_ITEMS
EMB = 32
HALF = 16
E = 1600000

NS = 16
EP = 1638400
EPT = EP // NS
C = 128
NCHUNK = EPT // C
NQ = NCHUNK // 4
PQ = EP // (4 * C)
NP = 100096
RPT = NP // NS
ZR = RPT // 17

_BLK = 2048


def _sc_propagate_body(ego_lo, ego_hi, src_ref, dst_ref, val_ref,
                       side_lo, side_hi,
                       qs0, qs1, qd0, qd1, qv0, qv1, r0, r1, r2, r3,
                       zbuf, acc,
                       sqs0, sqs1, sqd0, sqd1, sqv0, sqv1,
                       sg0, sg1, sg2, sg3, ss0, ss1, ss2, ss3):
    c = lax.axis_index("c")
    s = lax.axis_index("s")
    QS, QD, QV = [qs0, qs1], [qd0, qd1], [qv0, qv1]
    R = [r0, r1, r2, r3]
    SQS, SQD, SQV = [sqs0, sqs1], [sqd0, sqd1], [sqv0, sqv1]
    SG, SS = [sg0, sg1, sg2, sg3], [ss0, ss1, ss2, ss3]

    zero16 = jnp.zeros((HALF,), jnp.float32)

    def zrow(i, carry):
        zbuf[i, :] = zero16
        return carry
    lax.fori_loop(0, ZR, zrow, 0)

    row0 = s * RPT

    def zcp(k, carry):
        rr = pl.multiple_of(row0 + k * ZR, 8)
        pltpu.sync_copy(zbuf, acc.at[pl.ds(rr, ZR)])
        return carry
    lax.fori_loop(0, RPT // ZR, zcp, 0)
    plsc.subcore_barrier()

    qbase = s * NQ

    def edge_phase(tbl):
        def qstart(qid, b):
            pltpu.async_copy(src_ref.at[qid], QS[b], SQS[b])
            pltpu.async_copy(dst_ref.at[qid], QD[b], SQD[b])
            pltpu.async_copy(val_ref.at[qid], QV[b], SQV[b])

        def qwait(b):
            pltpu.make_async_copy(src_ref.at[0], QS[b], SQS[b]).wait()
            pltpu.make_async_copy(dst_ref.at[0], QD[b], SQD[b]).wait()
            pltpu.make_async_copy(val_ref.at[0], QV[b], SQV[b]).wait()

        def gs(b, j, r):
            pltpu.async_copy(tbl.at[QS[b].at[j]], R[r], SG[r])

        def gw(b, j, r):
            pltpu.make_async_copy(tbl.at[QS[b].at[j]], R[r], SG[r]).wait()

        def scale(b, j, r):
            def sc16(h, cr):
                o = pl.multiple_of(h * HALF, HALF)
                vv = QV[b][j, pl.ds(o, HALF)]
                for t in range(HALF):
                    R[r][o + t, :] = R[r][o + t, :] * vv[t]
                return cr
            lax.fori_loop(0, C // HALF, sc16, 0)

        def st(b, j, r):
            pltpu.async_copy(R[r], acc.at[QD[b].at[j]], SS[r], add=True)

        def sw(b, j, r):
            pltpu.make_async_copy(R[r], acc.at[QD[b].at[j]], SS[r]).wait()

        qstart(qbase + 0, 0)
        qstart(qbase + 1, 1)
        qwait(0)
        gs(0, 0, 0)
        gs(0, 1, 1)

        gs(0, 2, 2)
        gs(0, 3, 3)
        gw(0, 0, 0); scale(0, 0, 0); st(0, 0, 0)
        gw(0, 1, 1); scale(0, 1, 1); st(0, 1, 1)
        qwait(1)
        sw(0, 0, 0); gs(1, 0, 0)
        sw(0, 1, 1); gs(1, 1, 1)
        gw(0, 2, 2); scale(0, 2, 2); st(0, 2, 2)
        gw(0, 3, 3); scale(0, 3, 3); st(0, 3, 3)
        qstart(qbase + 2, 0)

        def body(m_id, a, nb):
            sw(a, 2, 2); gs(a, 2, 2)
            sw(a, 3, 3); gs(a, 3, 3)
            gw(a, 0, 0); scale(a, 0, 0); st(a, 0, 0)
            gw(a, 1, 1); scale(a, 1, 1); st(a, 1, 1)
            qwait(nb)
            sw(a, 0, 0); gs(nb, 0, 0)
            sw(a, 1, 1); gs(nb, 1, 1)
            gw(a, 2, 2); scale(a, 2, 2); st(a, 2, 2)
            gw(a, 3, 3); scale(a, 3, 3); st(a, 3, 3)
            qstart(m_id + 2, a)

        def pair(p, carry):
            m1 = qbase + 2 * p + 1
            body(m1, 1, 0)
            body(m1 + 1, 0, 1)
            return carry
        lax.fori_loop(0, (NQ - 2) // 2, pair, 0)

        sw(1, 2, 2); gs(1, 2, 2)
        sw(1, 3, 3); gs(1, 3, 3)
        gw(1, 0, 0); scale(1, 0, 0); st(1, 0, 0)
        gw(1, 1, 1); scale(1, 1, 1); st(1, 1, 1)
        gw(1, 2, 2); scale(1, 2, 2); st(1, 2, 2)
        gw(1, 3, 3); scale(1, 3, 3); st(1, 3, 3)
        sw(1, 0, 0)
        sw(1, 1, 1)
        sw(1, 2, 2)
        sw(1, 3, 3)
        qwait(0)

    @pl.when(c == 0)
    def _():
        edge_phase(ego_lo)

    @pl.when(c == 1)
    def _():
        edge_phase(ego_hi)

    plsc.subcore_barrier()

    @pl.when(c == 0)
    def _():
        def wb(k, carry):
            rr = pl.multiple_of(row0 + k * ZR, 8)
            pltpu.sync_copy(acc.at[pl.ds(rr, ZR)], zbuf)
            pltpu.sync_copy(zbuf, side_lo.at[pl.ds(rr, ZR)])
            return carry
        lax.fori_loop(0, RPT // ZR, wb, 0)

    @pl.when(c == 1)
    def _():
        def wb(k, carry):
            rr = pl.multiple_of(row0 + k * ZR, 8)
            pltpu.sync_copy(acc.at[pl.ds(rr, ZR)], zbuf)
            pltpu.sync_copy(zbuf, side_hi.at[pl.ds(rr, ZR)])
            return carry
        lax.fori_loop(0, RPT // ZR, wb, 0)


_sc_propagate = pl.kernel(
    _sc_propagate_body,
    out_type=(jax.ShapeDtypeStruct((NP, HALF), jnp.float32),
              jax.ShapeDtypeStruct((NP, HALF), jnp.float32)),
    mesh=plsc.VectorSubcoreMesh(core_axis_name="c", subcore_axis_name="s"),
    compiler_params=pltpu.CompilerParams(use_tc_tiling_on_sc=False),
    scratch_types=(
        [pltpu.VMEM((4, C), jnp.int32)] * 4
        + [pltpu.VMEM((4, C), jnp.float32)] * 2
        + [pltpu.VMEM((C, HALF), jnp.float32)] * 4
        + [
            pltpu.VMEM((ZR, HALF), jnp.float32),
            pltpu.VMEM_SHARED((NP, HALF), jnp.float32),
        ]
        + [pltpu.SemaphoreType.DMA] * 14
    ),
)


def _combine_body(slo_ref, shi_ref, ego_ref, wg_ref, bg_ref, wb_ref, bb_ref,
                  outf_ref, olo_ref, ohi_ref):
    side = jnp.concatenate([slo_ref[...], shi_ref[...]], axis=1)
    ego = ego_ref[...]
    s = jnp.dot(side, wg_ref[...], preferred_element_type=jnp.float32) + bg_ref[...]
    s = jnp.where(s >= 0, s, 0.01 * s)
    b = jnp.dot(ego * side, wb_ref[...], preferred_element_type=jnp.float32) + bb_ref[...]
    b = jnp.where(b >= 0, b, 0.01 * b)
    res = s + b
    outf_ref[...] = res
    olo_ref[...] = res[:, :HALF]
    ohi_ref[...] = res[:, HALF:]


def _combine(side_lo, side_hi, ego_flat, Wg, bg, Wb, bb):
    grid = (NP + _BLK - 1) // _BLK
    return pl.pallas_call(
        _combine_body,
        grid=(grid,),
        in_specs=[
            pl.BlockSpec((_BLK, HALF), lambda i: (i, 0)),
            pl.BlockSpec((_BLK, HALF), lambda i: (i, 0)),
            pl.BlockSpec((_BLK, EMB), lambda i: (i, 0)),
            pl.BlockSpec((EMB, EMB), lambda i: (0, 0)),
            pl.BlockSpec((1, EMB), lambda i: (0, 0)),
            pl.BlockSpec((EMB, EMB), lambda i: (0, 0)),
            pl.BlockSpec((1, EMB), lambda i: (0, 0)),
        ],
        out_specs=[
            pl.BlockSpec((_BLK, EMB), lambda i: (i, 0)),
            pl.BlockSpec((_BLK, HALF), lambda i: (i, 0)),
            pl.BlockSpec((_BLK, HALF), lambda i: (i, 0)),
        ],
        out_shape=[
            jax.ShapeDtypeStruct((NP, EMB), jnp.float32),
            jax.ShapeDtypeStruct((NP, HALF), jnp.float32),
            jax.ShapeDtypeStruct((NP, HALF), jnp.float32),
        ],
    )(side_lo, side_hi, ego_flat, Wg.T, bg.reshape(1, EMB),
      Wb.T, bb.reshape(1, EMB))


def kernel(user_indices, item_indices, adj_indices, adj_values, user_emb,
           item_emb, W_gc0, b_gc0, W_bi0, b_bi0, W_gc1, b_gc1, W_bi1, b_bi1):
    zpad16 = jnp.zeros((NP - N, HALF), jnp.float32)
    zpad32 = jnp.zeros((NP - N, EMB), jnp.float32)
    ego_lo = jnp.concatenate([user_emb[:, :HALF], item_emb[:, :HALF], zpad16])
    ego_hi = jnp.concatenate([user_emb[:, HALF:], item_emb[:, HALF:], zpad16])
    ego_flat = jnp.concatenate([user_emb, item_emb, zpad32])

    pad = EP - E
    srcr = jnp.pad(adj_indices[0], (0, pad + 4 * C)).reshape(PQ + 1, 4, C)
    dstr = jnp.pad(adj_indices[1], (0, pad + 4 * C)).reshape(PQ + 1, 4, C)
    valsr = jnp.pad(adj_values, (0, pad + 4 * C)).reshape(PQ + 1, 4, C)

    flats = [ego_flat]
    for (Wg, bg, Wb, bb) in ((W_gc0, b_gc0, W_bi0, b_bi0),
                             (W_gc1, b_gc1, W_bi1, b_bi1)):
        side_lo, side_hi = _sc_propagate(ego_lo, ego_hi, srcr, dstr, valsr)
        ego_flat, ego_lo, ego_hi = _combine(side_lo, side_hi, ego_flat,
                                            Wg, bg, Wb, bb)
        flats.append(ego_flat)
    all_c = jnp.concatenate([f[:N] for f in flats], axis=1)
    return (all_c[:NUM_USERS], all_c[NUM_USERS:])


# speedup vs baseline: 1.2408x; 1.1116x over previous
import jax
import jax.numpy as jnp
from jax import lax
from jax.experimental import pallas as pl
from jax.experimental.pallas import tpu as pltpu
from jax.experimental.pallas import tpu_sc as plsc

NUM_USERS = 50000
NUM_ITEMS = 50000
N = NUM_USERS + NUM_ITEMS
EMB = 32
HALF = 16
E = 1600000

NS = 16
EP = 1638400
EPT = EP // NS
C = 128
NCHUNK = EPT // C
NQ = NCHUNK // 4
PQ = EP // (4 * C)
NP = 100096
RPT = NP // NS
ZR = RPT // 17

_BLK = 2048


def _sc_propagate_body(ego_lo, ego_hi, src_ref, dst_ref, val_ref,
                       side_lo, side_hi,
                       qs0, qs1, qd0, qd1, qv0, qv1, r0, r1, r2, r3,
                       zbuf, acc,
                       sqs0, sqs1, sqd0, sqd1, sqv0, sqv1,
                       sg0, sg1, sg2, sg3, ss0, ss1, ss2, ss3):
    c = lax.axis_index("c")
    s = lax.axis_index("s")
    QS, QD, QV = [qs0, qs1], [qd0, qd1], [qv0, qv1]
    R = [r0, r1, r2, r3]
    SQS, SQD, SQV = [sqs0, sqs1], [sqd0, sqd1], [sqv0, sqv1]
    SG, SS = [sg0, sg1, sg2, sg3], [ss0, ss1, ss2, ss3]

    zero16 = jnp.zeros((HALF,), jnp.float32)

    def zrow(i, carry):
        zbuf[i, :] = zero16
        return carry
    lax.fori_loop(0, ZR, zrow, 0)

    row0 = s * RPT

    def zcp(k, carry):
        rr = pl.multiple_of(row0 + k * ZR, 8)
        pltpu.sync_copy(zbuf, acc.at[pl.ds(rr, ZR)])
        return carry
    lax.fori_loop(0, RPT // ZR, zcp, 0)
    plsc.subcore_barrier()

    qbase = s * NQ

    def edge_phase():
        def qstart(qid, b):
            pltpu.async_copy(src_ref.at[qid], QS[b], SQS[b])
            pltpu.async_copy(dst_ref.at[qid], QD[b], SQD[b])
            pltpu.async_copy(val_ref.at[qid], QV[b], SQV[b])

        def qwait(b):
            pltpu.make_async_copy(src_ref.at[0], QS[b], SQS[b]).wait()
            pltpu.make_async_copy(dst_ref.at[0], QD[b], SQD[b]).wait()
            pltpu.make_async_copy(val_ref.at[0], QV[b], SQV[b]).wait()

        def gs(b, j, r):
            @pl.when(c == 0)
            def _():
                pltpu.async_copy(ego_lo.at[QS[b].at[j]], R[r], SG[r])

            @pl.when(c == 1)
            def _():
                pltpu.async_copy(ego_hi.at[QS[b].at[j]], R[r], SG[r])

        def gw(b, j, r):
            @pl.when(c == 0)
            def _():
                pltpu.make_async_copy(ego_lo.at[QS[b].at[j]], R[r],
                                      SG[r]).wait()

            @pl.when(c == 1)
            def _():
                pltpu.make_async_copy(ego_hi.at[QS[b].at[j]], R[r],
                                      SG[r]).wait()

        def scale(b, j, r):
            def sc16(h, cr):
                o = pl.multiple_of(h * HALF, HALF)
                vv = QV[b][j, pl.ds(o, HALF)]
                for t in range(HALF):
                    R[r][o + t, :] = R[r][o + t, :] * vv[t]
                return cr
            lax.fori_loop(0, C // HALF, sc16, 0)

        def st(b, j, r):
            pltpu.async_copy(R[r], acc.at[QD[b].at[j]], SS[r], add=True)

        def sw(b, j, r):
            pltpu.make_async_copy(R[r], acc.at[QD[b].at[j]], SS[r]).wait()

        qstart(qbase + 0, 0)
        qstart(qbase + 1, 1)
        qwait(0)
        gs(0, 0, 0)
        gs(0, 1, 1)

        gs(0, 2, 2)
        gs(0, 3, 3)
        gw(0, 0, 0); scale(0, 0, 0); st(0, 0, 0)
        gw(0, 1, 1); scale(0, 1, 1); st(0, 1, 1)
        qwait(1)
        sw(0, 0, 0); gs(1, 0, 0)
        sw(0, 1, 1); gs(1, 1, 1)
        gw(0, 2, 2); scale(0, 2, 2); st(0, 2, 2)
        gw(0, 3, 3); scale(0, 3, 3); st(0, 3, 3)
        qstart(qbase + 2, 0)

        def body(m_id, a, nb):
            sw(a, 2, 2); gs(a, 2, 2)
            sw(a, 3, 3); gs(a, 3, 3)
            gw(a, 0, 0); scale(a, 0, 0); st(a, 0, 0)
            gw(a, 1, 1); scale(a, 1, 1); st(a, 1, 1)
            qwait(nb)
            sw(a, 0, 0); gs(nb, 0, 0)
            sw(a, 1, 1); gs(nb, 1, 1)
            gw(a, 2, 2); scale(a, 2, 2); st(a, 2, 2)
            gw(a, 3, 3); scale(a, 3, 3); st(a, 3, 3)
            qstart(m_id + 2, a)

        def pair(p, carry):
            m1 = qbase + 2 * p + 1
            body(m1, 1, 0)
            body(m1 + 1, 0, 1)
            return carry
        lax.fori_loop(0, (NQ - 2) // 2, pair, 0)

        sw(1, 2, 2); gs(1, 2, 2)
        sw(1, 3, 3); gs(1, 3, 3)
        gw(1, 0, 0); scale(1, 0, 0); st(1, 0, 0)
        gw(1, 1, 1); scale(1, 1, 1); st(1, 1, 1)
        gw(1, 2, 2); scale(1, 2, 2); st(1, 2, 2)
        gw(1, 3, 3); scale(1, 3, 3); st(1, 3, 3)
        sw(1, 0, 0)
        sw(1, 1, 1)
        sw(1, 2, 2)
        sw(1, 3, 3)
        qwait(0)

    edge_phase()
    plsc.subcore_barrier()

    @pl.when(c == 0)
    def _():
        def wb(k, carry):
            rr = pl.multiple_of(row0 + k * ZR, 8)
            pltpu.sync_copy(acc.at[pl.ds(rr, ZR)], zbuf)
            pltpu.sync_copy(zbuf, side_lo.at[pl.ds(rr, ZR)])
            return carry
        lax.fori_loop(0, RPT // ZR, wb, 0)

    @pl.when(c == 1)
    def _():
        def wb(k, carry):
            rr = pl.multiple_of(row0 + k * ZR, 8)
            pltpu.sync_copy(acc.at[pl.ds(rr, ZR)], zbuf)
            pltpu.sync_copy(zbuf, side_hi.at[pl.ds(rr, ZR)])
            return carry
        lax.fori_loop(0, RPT // ZR, wb, 0)


_sc_propagate = pl.kernel(
    _sc_propagate_body,
    out_type=(jax.ShapeDtypeStruct((NP, HALF), jnp.float32),
              jax.ShapeDtypeStruct((NP, HALF), jnp.float32)),
    mesh=plsc.VectorSubcoreMesh(core_axis_name="c", subcore_axis_name="s"),
    compiler_params=pltpu.CompilerParams(use_tc_tiling_on_sc=False),
    scratch_types=(
        [pltpu.VMEM((4, C), jnp.int32)] * 4
        + [pltpu.VMEM((4, C), jnp.float32)] * 2
        + [pltpu.VMEM((C, HALF), jnp.float32)] * 4
        + [
            pltpu.VMEM((ZR, HALF), jnp.float32),
            pltpu.VMEM_SHARED((NP, HALF), jnp.float32),
        ]
        + [pltpu.SemaphoreType.DMA] * 14
    ),
)


def _res(slo_ref, shi_ref, ego_ref, wg_ref, bg_ref, wb_ref, bb_ref):
    side = jnp.concatenate([slo_ref[...], shi_ref[...]], axis=1)
    ego = ego_ref[...]
    s = jnp.dot(side, wg_ref[...], preferred_element_type=jnp.float32) + bg_ref[...]
    s = jnp.where(s >= 0, s, 0.01 * s)
    b = jnp.dot(ego * side, wb_ref[...], preferred_element_type=jnp.float32) + bb_ref[...]
    b = jnp.where(b >= 0, b, 0.01 * b)
    return ego, s + b


def _combine_body(slo_ref, shi_ref, ego_ref, wg_ref, bg_ref, wb_ref, bb_ref,
                  outf_ref, olo_ref, ohi_ref):
    _, res = _res(slo_ref, shi_ref, ego_ref, wg_ref, bg_ref, wb_ref, bb_ref)
    outf_ref[...] = res
    olo_ref[...] = res[:, :HALF]
    ohi_ref[...] = res[:, HALF:]


def _combine_final_body(slo_ref, shi_ref, ego_ref, e0_ref,
                        wg_ref, bg_ref, wb_ref, bb_ref, allc_ref):
    ego, res = _res(slo_ref, shi_ref, ego_ref, wg_ref, bg_ref, wb_ref,
                    bb_ref)
    allc_ref[...] = jnp.concatenate([e0_ref[...], ego, res], axis=1)


def _combine_final(side_lo, side_hi, ego_flat, e0, Wg, bg, Wb, bb):
    grid = (NP + _BLK - 1) // _BLK
    return pl.pallas_call(
        _combine_final_body,
        grid=(grid,),
        in_specs=[
            pl.BlockSpec((_BLK, HALF), lambda i: (i, 0)),
            pl.BlockSpec((_BLK, HALF), lambda i: (i, 0)),
            pl.BlockSpec((_BLK, EMB), lambda i: (i, 0)),
            pl.BlockSpec((_BLK, EMB), lambda i: (i, 0)),
            pl.BlockSpec((EMB, EMB), lambda i: (0, 0)),
            pl.BlockSpec((1, EMB), lambda i: (0, 0)),
            pl.BlockSpec((EMB, EMB), lambda i: (0, 0)),
            pl.BlockSpec((1, EMB), lambda i: (0, 0)),
        ],
        out_specs=pl.BlockSpec((_BLK, 3 * EMB), lambda i: (i, 0)),
        out_shape=jax.ShapeDtypeStruct((NP, 3 * EMB), jnp.float32),
    )(side_lo, side_hi, ego_flat, e0, Wg.T, bg.reshape(1, EMB),
      Wb.T, bb.reshape(1, EMB))


def _combine(side_lo, side_hi, ego_flat, Wg, bg, Wb, bb):
    grid = (NP + _BLK - 1) // _BLK
    return pl.pallas_call(
        _combine_body,
        grid=(grid,),
        in_specs=[
            pl.BlockSpec((_BLK, HALF), lambda i: (i, 0)),
            pl.BlockSpec((_BLK, HALF), lambda i: (i, 0)),
            pl.BlockSpec((_BLK, EMB), lambda i: (i, 0)),
            pl.BlockSpec((EMB, EMB), lambda i: (0, 0)),
            pl.BlockSpec((1, EMB), lambda i: (0, 0)),
            pl.BlockSpec((EMB, EMB), lambda i: (0, 0)),
            pl.BlockSpec((1, EMB), lambda i: (0, 0)),
        ],
        out_specs=[
            pl.BlockSpec((_BLK, EMB), lambda i: (i, 0)),
            pl.BlockSpec((_BLK, HALF), lambda i: (i, 0)),
            pl.BlockSpec((_BLK, HALF), lambda i: (i, 0)),
        ],
        out_shape=[
            jax.ShapeDtypeStruct((NP, EMB), jnp.float32),
            jax.ShapeDtypeStruct((NP, HALF), jnp.float32),
            jax.ShapeDtypeStruct((NP, HALF), jnp.float32),
        ],
    )(side_lo, side_hi, ego_flat, Wg.T, bg.reshape(1, EMB),
      Wb.T, bb.reshape(1, EMB))


def kernel(user_indices, item_indices, adj_indices, adj_values, user_emb,
           item_emb, W_gc0, b_gc0, W_bi0, b_bi0, W_gc1, b_gc1, W_bi1, b_bi1):
    zpad16 = jnp.zeros((NP - N, HALF), jnp.float32)
    zpad32 = jnp.zeros((NP - N, EMB), jnp.float32)
    ego_lo = jnp.concatenate([user_emb[:, :HALF], item_emb[:, :HALF], zpad16])
    ego_hi = jnp.concatenate([user_emb[:, HALF:], item_emb[:, HALF:], zpad16])
    ego_flat = jnp.concatenate([user_emb, item_emb, zpad32])

    pad = EP - E
    srcr = jnp.pad(adj_indices[0], (0, pad + 4 * C)).reshape(PQ + 1, 4, C)
    dstr = jnp.pad(adj_indices[1], (0, pad + 4 * C)).reshape(PQ + 1, 4, C)
    valsr = jnp.pad(adj_values, (0, pad + 4 * C)).reshape(PQ + 1, 4, C)

    ego0_flat = ego_flat
    side_lo, side_hi = _sc_propagate(ego_lo, ego_hi, srcr, dstr, valsr)
    ego_flat, ego_lo, ego_hi = _combine(side_lo, side_hi, ego_flat,
                                        W_gc0, b_gc0, W_bi0, b_bi0)
    side_lo, side_hi = _sc_propagate(ego_lo, ego_hi, srcr, dstr, valsr)
    all_c = _combine_final(side_lo, side_hi, ego_flat, ego0_flat,
                           W_gc1, b_gc1, W_bi1, b_bi1)
    return (all_c[:NUM_USERS], all_c[NUM_USERS:N])


# speedup vs baseline: 1.2535x; 1.0102x over previous
import jax
import jax.numpy as jnp
from jax import lax
from jax.experimental import pallas as pl
from jax.experimental.pallas import tpu as pltpu
from jax.experimental.pallas import tpu_sc as plsc

NUM_USERS = 50000
NUM_ITEMS = 50000
N = NUM_USERS + NUM_ITEMS
EMB = 32
HALF = 16
E = 1600000

NS = 16
EP = 1638400
EPT = EP // NS
C = 128
NCHUNK = EPT // C
NQ = NCHUNK // 4
PQ = EP // (4 * C)
NP = 100096
RPT = NP // NS
ZR = RPT // 17

_BLK = 4096


def _sc_propagate_body(ego_lo, ego_hi, src_ref, dst_ref, val_ref,
                       side_lo, side_hi,
                       qs0, qs1, qd0, qd1, qv0, qv1, r0, r1, r2, r3,
                       zbuf, acc,
                       sqs0, sqs1, sqd0, sqd1, sqv0, sqv1,
                       sg0, sg1, sg2, sg3, ss0, ss1, ss2, ss3):
    c = lax.axis_index("c")
    s = lax.axis_index("s")
    QS, QD, QV = [qs0, qs1], [qd0, qd1], [qv0, qv1]
    R = [r0, r1, r2, r3]
    SQS, SQD, SQV = [sqs0, sqs1], [sqd0, sqd1], [sqv0, sqv1]
    SG, SS = [sg0, sg1, sg2, sg3], [ss0, ss1, ss2, ss3]

    zero16 = jnp.zeros((HALF,), jnp.float32)

    def zrow(i, carry):
        zbuf[i, :] = zero16
        return carry
    lax.fori_loop(0, ZR, zrow, 0)

    row0 = s * RPT

    def zcp(k, carry):
        rr = pl.multiple_of(row0 + k * ZR, 8)
        pltpu.sync_copy(zbuf, acc.at[pl.ds(rr, ZR)])
        return carry
    lax.fori_loop(0, RPT // ZR, zcp, 0)
    plsc.subcore_barrier()

    qbase = s * NQ

    def edge_phase():
        def qstart(qid, b):
            pltpu.async_copy(src_ref.at[qid], QS[b], SQS[b])
            pltpu.async_copy(dst_ref.at[qid], QD[b], SQD[b])
            pltpu.async_copy(val_ref.at[qid], QV[b], SQV[b])

        def qwait(b):
            pltpu.make_async_copy(src_ref.at[0], QS[b], SQS[b]).wait()
            pltpu.make_async_copy(dst_ref.at[0], QD[b], SQD[b]).wait()
            pltpu.make_async_copy(val_ref.at[0], QV[b], SQV[b]).wait()

        def gs(b, j, r):
            @pl.when(c == 0)
            def _():
                pltpu.async_copy(ego_lo.at[QS[b].at[j]], R[r], SG[r])

            @pl.when(c == 1)
            def _():
                pltpu.async_copy(ego_hi.at[QS[b].at[j]], R[r], SG[r])

        def gw(b, j, r):
            @pl.when(c == 0)
            def _():
                pltpu.make_async_copy(ego_lo.at[QS[b].at[j]], R[r],
                                      SG[r]).wait()

            @pl.when(c == 1)
            def _():
                pltpu.make_async_copy(ego_hi.at[QS[b].at[j]], R[r],
                                      SG[r]).wait()

        def scale(b, j, r):
            def sc16(h, cr):
                o = pl.multiple_of(h * HALF, HALF)
                vv = QV[b][j, pl.ds(o, HALF)]
                for t in range(HALF):
                    R[r][o + t, :] = R[r][o + t, :] * vv[t]
                return cr
            lax.fori_loop(0, C // HALF, sc16, 0)

        def st(b, j, r):
            pltpu.async_copy(R[r], acc.at[QD[b].at[j]], SS[r], add=True)

        def sw(b, j, r):
            pltpu.make_async_copy(R[r], acc.at[QD[b].at[j]], SS[r]).wait()

        qstart(qbase + 0, 0)
        qstart(qbase + 1, 1)
        qwait(0)
        gs(0, 0, 0)
        gs(0, 1, 1)

        gs(0, 2, 2)
        gs(0, 3, 3)
        gw(0, 0, 0); scale(0, 0, 0); st(0, 0, 0)
        gw(0, 1, 1); scale(0, 1, 1); st(0, 1, 1)
        qwait(1)
        sw(0, 0, 0); gs(1, 0, 0)
        sw(0, 1, 1); gs(1, 1, 1)
        gw(0, 2, 2); scale(0, 2, 2); st(0, 2, 2)
        gw(0, 3, 3); scale(0, 3, 3); st(0, 3, 3)
        qstart(qbase + 2, 0)

        def body(m_id, a, nb):
            sw(a, 2, 2); gs(a, 2, 2)
            sw(a, 3, 3); gs(a, 3, 3)
            gw(a, 0, 0); scale(a, 0, 0); st(a, 0, 0)
            gw(a, 1, 1); scale(a, 1, 1); st(a, 1, 1)
            qwait(nb)
            sw(a, 0, 0); gs(nb, 0, 0)
            sw(a, 1, 1); gs(nb, 1, 1)
            gw(a, 2, 2); scale(a, 2, 2); st(a, 2, 2)
            gw(a, 3, 3); scale(a, 3, 3); st(a, 3, 3)
            qstart(m_id + 2, a)

        def pair(p, carry):
            m1 = qbase + 2 * p + 1
            body(m1, 1, 0)
            body(m1 + 1, 0, 1)
            return carry
        lax.fori_loop(0, (NQ - 2) // 2, pair, 0)

        sw(1, 2, 2); gs(1, 2, 2)
        sw(1, 3, 3); gs(1, 3, 3)
        gw(1, 0, 0); scale(1, 0, 0); st(1, 0, 0)
        gw(1, 1, 1); scale(1, 1, 1); st(1, 1, 1)
        gw(1, 2, 2); scale(1, 2, 2); st(1, 2, 2)
        gw(1, 3, 3); scale(1, 3, 3); st(1, 3, 3)
        sw(1, 0, 0)
        sw(1, 1, 1)
        sw(1, 2, 2)
        sw(1, 3, 3)
        qwait(0)

    edge_phase()
    plsc.subcore_barrier()

    @pl.when(c == 0)
    def _():
        def wb(k, carry):
            rr = pl.multiple_of(row0 + k * ZR, 8)
            pltpu.sync_copy(acc.at[pl.ds(rr, ZR)], zbuf)
            pltpu.sync_copy(zbuf, side_lo.at[pl.ds(rr, ZR)])
            return carry
        lax.fori_loop(0, RPT // ZR, wb, 0)

    @pl.when(c == 1)
    def _():
        def wb(k, carry):
            rr = pl.multiple_of(row0 + k * ZR, 8)
            pltpu.sync_copy(acc.at[pl.ds(rr, ZR)], zbuf)
            pltpu.sync_copy(zbuf, side_hi.at[pl.ds(rr, ZR)])
            return carry
        lax.fori_loop(0, RPT // ZR, wb, 0)


_sc_propagate = pl.kernel(
    _sc_propagate_body,
    out_type=(jax.ShapeDtypeStruct((NP, HALF), jnp.float32),
              jax.ShapeDtypeStruct((NP, HALF), jnp.float32)),
    mesh=plsc.VectorSubcoreMesh(core_axis_name="c", subcore_axis_name="s"),
    compiler_params=pltpu.CompilerParams(use_tc_tiling_on_sc=False),
    scratch_types=(
        [pltpu.VMEM((4, C), jnp.int32)] * 4
        + [pltpu.VMEM((4, C), jnp.float32)] * 2
        + [pltpu.VMEM((C, HALF), jnp.float32)] * 4
        + [
            pltpu.VMEM((ZR, HALF), jnp.float32),
            pltpu.VMEM_SHARED((NP, HALF), jnp.float32),
        ]
        + [pltpu.SemaphoreType.DMA] * 14
    ),
)


def _res(slo_ref, shi_ref, ego_ref, wg_ref, bg_ref, wb_ref, bb_ref):
    side = jnp.concatenate([slo_ref[...], shi_ref[...]], axis=1)
    ego = ego_ref[...]
    s = jnp.dot(side, wg_ref[...], preferred_element_type=jnp.float32) + bg_ref[...]
    s = jnp.where(s >= 0, s, 0.01 * s)
    b = jnp.dot(ego * side, wb_ref[...], preferred_element_type=jnp.float32) + bb_ref[...]
    b = jnp.where(b >= 0, b, 0.01 * b)
    return ego, s + b


def _combine_body(slo_ref, shi_ref, ego_ref, wg_ref, bg_ref, wb_ref, bb_ref,
                  outf_ref, olo_ref, ohi_ref):
    _, res = _res(slo_ref, shi_ref, ego_ref, wg_ref, bg_ref, wb_ref, bb_ref)
    outf_ref[...] = res
    olo_ref[...] = res[:, :HALF]
    ohi_ref[...] = res[:, HALF:]


def _combine_final_body(slo_ref, shi_ref, ego_ref, e0_ref,
                        wg_ref, bg_ref, wb_ref, bb_ref, allc_ref):
    ego, res = _res(slo_ref, shi_ref, ego_ref, wg_ref, bg_ref, wb_ref,
                    bb_ref)
    allc_ref[...] = jnp.concatenate([e0_ref[...], ego, res], axis=1)


def _combine_final(side_lo, side_hi, ego_flat, e0, Wg, bg, Wb, bb):
    grid = (NP + _BLK - 1) // _BLK
    return pl.pallas_call(
        _combine_final_body,
        grid=(grid,),
        in_specs=[
            pl.BlockSpec((_BLK, HALF), lambda i: (i, 0)),
            pl.BlockSpec((_BLK, HALF), lambda i: (i, 0)),
            pl.BlockSpec((_BLK, EMB), lambda i: (i, 0)),
            pl.BlockSpec((_BLK, EMB), lambda i: (i, 0)),
            pl.BlockSpec((EMB, EMB), lambda i: (0, 0)),
            pl.BlockSpec((1, EMB), lambda i: (0, 0)),
            pl.BlockSpec((EMB, EMB), lambda i: (0, 0)),
            pl.BlockSpec((1, EMB), lambda i: (0, 0)),
        ],
        out_specs=pl.BlockSpec((_BLK, 3 * EMB), lambda i: (i, 0)),
        out_shape=jax.ShapeDtypeStruct((NP, 3 * EMB), jnp.float32),
    )(side_lo, side_hi, ego_flat, e0, Wg.T, bg.reshape(1, EMB),
      Wb.T, bb.reshape(1, EMB))


def _combine(side_lo, side_hi, ego_flat, Wg, bg, Wb, bb):
    grid = (NP + _BLK - 1) // _BLK
    return pl.pallas_call(
        _combine_body,
        grid=(grid,),
        in_specs=[
            pl.BlockSpec((_BLK, HALF), lambda i: (i, 0)),
            pl.BlockSpec((_BLK, HALF), lambda i: (i, 0)),
            pl.BlockSpec((_BLK, EMB), lambda i: (i, 0)),
            pl.BlockSpec((EMB, EMB), lambda i: (0, 0)),
            pl.BlockSpec((1, EMB), lambda i: (0, 0)),
            pl.BlockSpec((EMB, EMB), lambda i: (0, 0)),
            pl.BlockSpec((1, EMB), lambda i: (0, 0)),
        ],
        out_specs=[
            pl.BlockSpec((_BLK, EMB), lambda i: (i, 0)),
            pl.BlockSpec((_BLK, HALF), lambda i: (i, 0)),
            pl.BlockSpec((_BLK, HALF), lambda i: (i, 0)),
        ],
        out_shape=[
            jax.ShapeDtypeStruct((NP, EMB), jnp.float32),
            jax.ShapeDtypeStruct((NP, HALF), jnp.float32),
            jax.ShapeDtypeStruct((NP, HALF), jnp.float32),
        ],
    )(side_lo, side_hi, ego_flat, Wg.T, bg.reshape(1, EMB),
      Wb.T, bb.reshape(1, EMB))


def kernel(user_indices, item_indices, adj_indices, adj_values, user_emb,
           item_emb, W_gc0, b_gc0, W_bi0, b_bi0, W_gc1, b_gc1, W_bi1, b_bi1):
    zpad16 = jnp.zeros((NP - N, HALF), jnp.float32)
    zpad32 = jnp.zeros((NP - N, EMB), jnp.float32)
    ego_lo = jnp.concatenate([user_emb[:, :HALF], item_emb[:, :HALF], zpad16])
    ego_hi = jnp.concatenate([user_emb[:, HALF:], item_emb[:, HALF:], zpad16])
    ego_flat = jnp.concatenate([user_emb, item_emb, zpad32])

    pad = EP - E
    srcr = jnp.pad(adj_indices[0], (0, pad + 4 * C)).reshape(PQ + 1, 4, C)
    dstr = jnp.pad(adj_indices[1], (0, pad + 4 * C)).reshape(PQ + 1, 4, C)
    valsr = jnp.pad(adj_values, (0, pad + 4 * C)).reshape(PQ + 1, 4, C)

    ego0_flat = ego_flat
    side_lo, side_hi = _sc_propagate(ego_lo, ego_hi, srcr, dstr, valsr)
    ego_flat, ego_lo, ego_hi = _combine(side_lo, side_hi, ego_flat,
                                        W_gc0, b_gc0, W_bi0, b_bi0)
    side_lo, side_hi = _sc_propagate(ego_lo, ego_hi, srcr, dstr, valsr)
    all_c = _combine_final(side_lo, side_hi, ego_flat, ego0_flat,
                           W_gc1, b_gc1, W_bi1, b_bi1)
    return (all_c[:NUM_USERS], all_c[NUM_USERS:N])
